# baseline (speedup 1.0000x reference)
import functools

import jax
import jax.numpy as jnp
from jax import lax
from jax.experimental import pallas as pl
from jax.experimental.pallas import tpu as pltpu
from jax.experimental.pallas import tpu_sc as plsc

_HEADS = 10
_K = 10
_NWORK = 32


def _cdiv(a, b):
    return (a + b - 1) // b


def _sc_gather(table, idx):
    _, f = table.shape
    B = idx.shape[0]
    CH = 512
    chunk_rows = _NWORK * CH
    n_chunks = _cdiv(B, chunk_rows)
    Bp = n_chunks * chunk_rows
    idx_p = jnp.pad(idx, (0, Bp - B))
    per_w = n_chunks * CH

    mesh = plsc.VectorSubcoreMesh(core_axis_name="c", subcore_axis_name="s")

    @functools.partial(
        pl.kernel,
        mesh=mesh,
        out_type=jax.ShapeDtypeStruct((Bp, f), jnp.float32),
        scratch_types=[
            pltpu.VMEM((CH,), jnp.int32),
            pltpu.VMEM((CH, f), jnp.float32),
            pltpu.SemaphoreType.DMA,
        ],
        compiler_params=pltpu.CompilerParams(use_tc_tiling_on_sc=False),
    )
    def gk(table_hbm, idx_hbm, out_hbm, idx_v, rows_v, sem):
        wid = lax.axis_index("s") * 2 + lax.axis_index("c")
        base = wid * per_w
        for c in range(n_chunks):
            off = base + c * CH
            pltpu.sync_copy(idx_hbm.at[pl.ds(off, CH)], idx_v)
            pltpu.async_copy(table_hbm.at[idx_v], rows_v, sem).wait()
            pltpu.sync_copy(rows_v, out_hbm.at[pl.ds(off, CH)])

    return gk(table, idx_p)[:B]


def _feast_attn(xn2, n, b, f, out_c, WmT, bm2, WoT, bias2, act):
    rows = b * n
    NB = min(512, _cdiv(rows, 8) * 8)
    grid = (_cdiv(rows, NB),)

    def kern(xn_ref, wm_ref, bm_ref, wo_ref, bias_ref, o_ref):
        xb = xn_ref[...]
        x0 = xb[:, 0:f]
        qs = []
        for k in range(_K):
            xk = xb[:, k * f:(k + 1) * f]
            lg = jnp.dot(xk - x0, wm_ref[...],
                         preferred_element_type=jnp.float32) + bm_ref[...]
            lg = lg - jnp.max(lg, axis=-1, keepdims=True)
            e = jnp.exp(lg)
            qs.append(e / jnp.sum(e, axis=-1, keepdims=True))
        acc = jnp.zeros((NB, out_c), jnp.float32)
        for h in range(_HEADS):
            wh = qs[0][:, h:h + 1] * xb[:, 0:f]
            for k in range(1, _K):
                wh = wh + qs[k][:, h:h + 1] * xb[:, k * f:(k + 1) * f]
            acc = acc + jnp.dot(wh, wo_ref[h * f:(h + 1) * f, :],
                                preferred_element_type=jnp.float32)
        out = acc + bias_ref[...]
        gid = pl.program_id(0) * NB + lax.broadcasted_iota(jnp.int32, (NB, 1), 0)
        out = jnp.where((gid % n) == (n - 1), 0.0, out)
        if act:
            out = jnp.where(out > 0, out, jnp.exp(out) - 1.0)
        o_ref[...] = out

    return pl.pallas_call(
        kern,
        grid=grid,
        in_specs=[
            pl.BlockSpec((NB, _K * f), lambda i: (i, 0)),
            pl.BlockSpec(WmT.shape, lambda i: (0, 0)),
            pl.BlockSpec(bm2.shape, lambda i: (0, 0)),
            pl.BlockSpec(WoT.shape, lambda i: (0, 0)),
            pl.BlockSpec(bias2.shape, lambda i: (0, 0)),
        ],
        out_specs=pl.BlockSpec((NB, out_c), lambda i: (i, 0)),
        out_shape=jax.ShapeDtypeStruct((rows, out_c), jnp.float32),
    )(xn2, WmT, bm2, WoT, bias2)


def _matmul(A, B, bias2):
    M, Kd = A.shape
    _, N = B.shape
    BM = min(256, M)
    BN = min(512, N)
    BK = Kd if Kd <= 2560 else 2048
    grid = (_cdiv(M, BM), _cdiv(N, BN), _cdiv(Kd, BK))

    def kern(a_ref, b_ref, bias_ref, o_ref):
        @pl.when(pl.program_id(2) == 0)
        def _():
            o_ref[...] = jnp.zeros_like(o_ref)

        a = a_ref[...]
        if Kd % BK != 0:
            kbase = pl.program_id(2) * BK
            lane = lax.broadcasted_iota(jnp.int32, (BM, BK), 1)
            a = jnp.where(lane + kbase < Kd, a, 0.0)
        o_ref[...] += jnp.dot(a, b_ref[...],
                              preferred_element_type=jnp.float32)

        @pl.when(pl.program_id(2) == pl.num_programs(2) - 1)
        def _():
            o_ref[...] += bias_ref[...]

    return pl.pallas_call(
        kern,
        grid=grid,
        in_specs=[
            pl.BlockSpec((BM, BK), lambda i, j, k: (i, k)),
            pl.BlockSpec((BK, BN), lambda i, j, k: (k, j)),
            pl.BlockSpec((1, BN), lambda i, j, k: (0, j)),
        ],
        out_specs=pl.BlockSpec((BM, BN), lambda i, j, k: (i, j)),
        out_shape=jax.ShapeDtypeStruct((M, N), jnp.float32),
    )(A, B, bias2)


def _prep_conv(p, in_c, f_pad, out_c):
    Wm = p["Wm"]
    Wo = p["Wo"]
    WmT = jnp.pad(Wm, ((0, 0), (0, f_pad - in_c))).T
    WoT = jnp.pad(
        Wo.reshape(_HEADS, out_c, in_c).transpose(0, 2, 1),
        ((0, 0), (0, f_pad - in_c), (0, 0)),
    ).reshape(_HEADS * f_pad, out_c)
    return WmT, p["bm"][None, :], WoT, p["bias"][None, :]


def _feast_layer(h, Si, p, out_c, act):
    b, n, f = h.shape
    f_pad = max(16, f)
    in_c = f
    if f_pad != f:
        h = jnp.pad(h, ((0, 0), (0, 0), (0, f_pad - f)))
        f = f_pad
    WmT, bm2, WoT, bias2 = _prep_conv(p, in_c, f_pad, out_c)
    table = h.reshape(b * n, f)
    offs = (jnp.arange(b, dtype=jnp.int32) * n)[:, None]
    idx = (Si.reshape(-1)[None, :] + offs).reshape(-1)
    xn = _sc_gather(table, idx)
    xn2 = xn.reshape(b * n, _K * f)
    out = _feast_attn(xn2, n, b, f, out_c, WmT, bm2, WoT, bias2, act)
    return out.reshape(b, n, out_c)


def _pool(L, h):
    b, m, f = h.shape
    p = L.shape[0]
    h2 = h.transpose(1, 0, 2).reshape(m, b * f)
    zb = jnp.zeros((1, b * f), jnp.float32)
    out2 = _matmul(L, h2, zb)
    return out2.reshape(p, b, f).transpose(1, 0, 2)


def kernel(x, S, D, U, theta):
    c = theta["convs"]
    b = x.shape[0]
    h = _feast_layer(x, S[0], c[0], 32, True)
    h = _pool(D[0], h)
    h = _feast_layer(h, S[1], c[1], 32, True)
    h = _pool(D[1], h)
    h = _feast_layer(h, S[2], c[2], 64, True)
    h = _pool(D[2], h)
    nf = h.shape[1] * h.shape[2]
    z = _matmul(h.reshape(b, nf), theta["We"].T, theta["be"][None, :])
    hd = _matmul(z, theta["Wd"].T, theta["bd"][None, :])
    h = hd.reshape(b, -1, 64)
    h = _pool(U[2], h)
    h = _feast_layer(h, S[2], c[3], 64, True)
    h = _pool(U[1], h)
    h = _feast_layer(h, S[1], c[4], 32, True)
    h = _pool(U[0], h)
    h = _feast_layer(h, S[0], c[5], 32, True)
    h = _feast_layer(h, S[0], c[6], 3, False)
    return h


# speedup vs baseline: 1.2673x; 1.2673x over previous
import functools

import jax
import jax.numpy as jnp
from jax import lax
from jax.experimental import pallas as pl
from jax.experimental.pallas import tpu as pltpu
from jax.experimental.pallas import tpu_sc as plsc

_HEADS = 10
_K = 10
_NWORK = 32


def _cdiv(a, b):
    return (a + b - 1) // b


def _sc_gather(table, idx):
    _, f = table.shape
    B = idx.shape[0]
    CH = 512 if f > 32 else 1024
    per_w = _cdiv(_cdiv(B, _NWORK), 8) * 8
    Bp = _NWORK * per_w
    n_chunks = _cdiv(per_w, CH)
    idx_p = jnp.pad(idx, (0, Bp - B))

    mesh = plsc.VectorSubcoreMesh(core_axis_name="c", subcore_axis_name="s")

    @functools.partial(
        pl.kernel,
        mesh=mesh,
        out_type=jax.ShapeDtypeStruct((Bp, f), jnp.float32),
        scratch_types=[
            pltpu.VMEM((per_w,), jnp.int32),
            pltpu.VMEM((CH, f), jnp.float32),
            pltpu.VMEM((CH, f), jnp.float32),
            pltpu.SemaphoreType.DMA,
            pltpu.SemaphoreType.DMA,
            pltpu.SemaphoreType.DMA,
            pltpu.SemaphoreType.DMA,
        ],
        compiler_params=pltpu.CompilerParams(use_tc_tiling_on_sc=False),
    )
    def gk(table_hbm, idx_hbm, out_hbm, idx_v, buf0, buf1, g0, g1, s0, s1):
        wid = lax.axis_index("s") * 2 + lax.axis_index("c")
        base = wid * per_w
        pltpu.sync_copy(idx_hbm.at[pl.ds(base, per_w)], idx_v)
        bufs = (buf0, buf1)
        gsems = (g0, g1)
        ssems = (s0, s1)

        def sz(c):
            return CH if (c + 1) * CH <= per_w else per_w - c * CH

        def start_gather(c):
            return pltpu.async_copy(
                table_hbm.at[idx_v.at[pl.ds(c * CH, sz(c))]],
                bufs[c % 2].at[pl.ds(0, sz(c))],
                gsems[c % 2],
            )

        def start_scatter(c):
            return pltpu.async_copy(
                bufs[c % 2].at[pl.ds(0, sz(c))],
                out_hbm.at[pl.ds(base + c * CH, sz(c))],
                ssems[c % 2],
            )

        gh = {}
        sh = {}
        for c in range(n_chunks):
            if c >= 2:
                sh[c - 2].wait()
            gh[c] = start_gather(c)
            if c >= 1:
                gh[c - 1].wait()
                sh[c - 1] = start_scatter(c - 1)
        gh[n_chunks - 1].wait()
        sh[n_chunks - 1] = start_scatter(n_chunks - 1)
        if n_chunks >= 2:
            sh[n_chunks - 2].wait()
        sh[n_chunks - 1].wait()

    return gk(table, idx_p)[:B]


def _feast_attn(xn2, n, b, f, out_c, WmT, bm2, WoT, bias2, act):
    rows = b * n
    NB = min(512, _cdiv(rows, 8) * 8)
    grid = (_cdiv(rows, NB),)

    def kern(xn_ref, wm_ref, bm_ref, wo_ref, bias_ref, o_ref):
        xb = xn_ref[...]
        x0 = xb[:, 0:f]
        qs = []
        for k in range(_K):
            xk = xb[:, k * f:(k + 1) * f]
            lg = jnp.dot(xk - x0, wm_ref[...],
                         preferred_element_type=jnp.float32) + bm_ref[...]
            lg = lg - jnp.max(lg, axis=-1, keepdims=True)
            e = jnp.exp(lg)
            qs.append(e / jnp.sum(e, axis=-1, keepdims=True))
        acc = jnp.zeros((NB, out_c), jnp.float32)
        for h in range(_HEADS):
            wh = qs[0][:, h:h + 1] * xb[:, 0:f]
            for k in range(1, _K):
                wh = wh + qs[k][:, h:h + 1] * xb[:, k * f:(k + 1) * f]
            acc = acc + jnp.dot(wh, wo_ref[h * f:(h + 1) * f, :],
                                preferred_element_type=jnp.float32)
        out = acc + bias_ref[...]
        gid = pl.program_id(0) * NB + lax.broadcasted_iota(jnp.int32, (NB, 1), 0)
        out = jnp.where((gid % n) == (n - 1), 0.0, out)
        if act:
            out = jnp.where(out > 0, out, jnp.exp(out) - 1.0)
        o_ref[...] = out

    return pl.pallas_call(
        kern,
        grid=grid,
        in_specs=[
            pl.BlockSpec((NB, _K * f), lambda i: (i, 0)),
            pl.BlockSpec(WmT.shape, lambda i: (0, 0)),
            pl.BlockSpec(bm2.shape, lambda i: (0, 0)),
            pl.BlockSpec(WoT.shape, lambda i: (0, 0)),
            pl.BlockSpec(bias2.shape, lambda i: (0, 0)),
        ],
        out_specs=pl.BlockSpec((NB, out_c), lambda i: (i, 0)),
        out_shape=jax.ShapeDtypeStruct((rows, out_c), jnp.float32),
    )(xn2, WmT, bm2, WoT, bias2)


def _matmul(A, B, bias2):
    M, Kd = A.shape
    _, N = B.shape
    BM = min(256, M)
    BN = min(512, N)
    BK = Kd if Kd <= 2560 else 2048
    grid = (_cdiv(M, BM), _cdiv(N, BN), _cdiv(Kd, BK))

    def kern(a_ref, b_ref, bias_ref, o_ref):
        @pl.when(pl.program_id(2) == 0)
        def _():
            o_ref[...] = jnp.zeros_like(o_ref)

        a = a_ref[...]
        if Kd % BK != 0:
            kbase = pl.program_id(2) * BK
            lane = lax.broadcasted_iota(jnp.int32, (BM, BK), 1)
            a = jnp.where(lane + kbase < Kd, a, 0.0)
        o_ref[...] += jnp.dot(a, b_ref[...],
                              preferred_element_type=jnp.float32)

        @pl.when(pl.program_id(2) == pl.num_programs(2) - 1)
        def _():
            o_ref[...] += bias_ref[...]

    return pl.pallas_call(
        kern,
        grid=grid,
        in_specs=[
            pl.BlockSpec((BM, BK), lambda i, j, k: (i, k)),
            pl.BlockSpec((BK, BN), lambda i, j, k: (k, j)),
            pl.BlockSpec((1, BN), lambda i, j, k: (0, j)),
        ],
        out_specs=pl.BlockSpec((BM, BN), lambda i, j, k: (i, j)),
        out_shape=jax.ShapeDtypeStruct((M, N), jnp.float32),
    )(A, B, bias2)


def _prep_conv(p, in_c, f_pad, out_c):
    Wm = p["Wm"]
    Wo = p["Wo"]
    WmT = jnp.pad(Wm, ((0, 0), (0, f_pad - in_c))).T
    WoT = jnp.pad(
        Wo.reshape(_HEADS, out_c, in_c).transpose(0, 2, 1),
        ((0, 0), (0, f_pad - in_c), (0, 0)),
    ).reshape(_HEADS * f_pad, out_c)
    return WmT, p["bm"][None, :], WoT, p["bias"][None, :]


def _feast_layer(h, Si, p, out_c, act):
    b, n, f = h.shape
    f_pad = max(16, f)
    in_c = f
    if f_pad != f:
        h = jnp.pad(h, ((0, 0), (0, 0), (0, f_pad - f)))
        f = f_pad
    WmT, bm2, WoT, bias2 = _prep_conv(p, in_c, f_pad, out_c)
    table = h.reshape(b * n, f)
    offs = (jnp.arange(b, dtype=jnp.int32) * n)[:, None]
    idx = (Si.reshape(-1)[None, :] + offs).reshape(-1)
    xn = _sc_gather(table, idx)
    xn2 = xn.reshape(b * n, _K * f)
    out = _feast_attn(xn2, n, b, f, out_c, WmT, bm2, WoT, bias2, act)
    return out.reshape(b, n, out_c)


def _pool(L, h):
    b, m, f = h.shape
    p = L.shape[0]
    h2 = h.transpose(1, 0, 2).reshape(m, b * f)
    zb = jnp.zeros((1, b * f), jnp.float32)
    out2 = _matmul(L, h2, zb)
    return out2.reshape(p, b, f).transpose(1, 0, 2)


def kernel(x, S, D, U, theta):
    c = theta["convs"]
    b = x.shape[0]
    h = _feast_layer(x, S[0], c[0], 32, True)
    h = _pool(D[0], h)
    h = _feast_layer(h, S[1], c[1], 32, True)
    h = _pool(D[1], h)
    h = _feast_layer(h, S[2], c[2], 64, True)
    h = _pool(D[2], h)
    nf = h.shape[1] * h.shape[2]
    z = _matmul(h.reshape(b, nf), theta["We"].T, theta["be"][None, :])
    hd = _matmul(z, theta["Wd"].T, theta["bd"][None, :])
    h = hd.reshape(b, -1, 64)
    h = _pool(U[2], h)
    h = _feast_layer(h, S[2], c[3], 64, True)
    h = _pool(U[1], h)
    h = _feast_layer(h, S[1], c[4], 32, True)
    h = _pool(U[0], h)
    h = _feast_layer(h, S[0], c[5], 32, True)
    h = _feast_layer(h, S[0], c[6], 3, False)
    return h


# speedup vs baseline: 2.1662x; 1.7093x over previous
import functools

import jax
import jax.numpy as jnp
from jax import lax
from jax.experimental import pallas as pl
from jax.experimental.pallas import tpu as pltpu
from jax.experimental.pallas import tpu_sc as plsc

_HEADS = 10
_K = 10
_NWORK = 32


def _cdiv(a, b):
    return (a + b - 1) // b


def _sc_gather(table, idx):
    _, f = table.shape
    B = idx.shape[0]
    CH = 512 if f > 32 else 1024
    per_w = _cdiv(_cdiv(B, _NWORK), 8) * 8
    Bp = _NWORK * per_w
    n_chunks = _cdiv(per_w, CH)
    idx_p = jnp.pad(idx, (0, Bp - B))

    mesh = plsc.VectorSubcoreMesh(core_axis_name="c", subcore_axis_name="s")

    @functools.partial(
        pl.kernel,
        mesh=mesh,
        out_type=jax.ShapeDtypeStruct((Bp, f), jnp.float32),
        scratch_types=[
            pltpu.VMEM((per_w,), jnp.int32),
            pltpu.VMEM((CH, f), jnp.float32),
            pltpu.VMEM((CH, f), jnp.float32),
            pltpu.SemaphoreType.DMA,
            pltpu.SemaphoreType.DMA,
            pltpu.SemaphoreType.DMA,
            pltpu.SemaphoreType.DMA,
        ],
        compiler_params=pltpu.CompilerParams(use_tc_tiling_on_sc=False),
    )
    def gk(table_hbm, idx_hbm, out_hbm, idx_v, buf0, buf1, g0, g1, s0, s1):
        wid = lax.axis_index("s") * 2 + lax.axis_index("c")
        base = wid * per_w
        pltpu.sync_copy(idx_hbm.at[pl.ds(base, per_w)], idx_v)
        bufs = (buf0, buf1)
        gsems = (g0, g1)
        ssems = (s0, s1)

        def sz(c):
            return CH if (c + 1) * CH <= per_w else per_w - c * CH

        def start_gather(c):
            return pltpu.async_copy(
                table_hbm.at[idx_v.at[pl.ds(c * CH, sz(c))]],
                bufs[c % 2].at[pl.ds(0, sz(c))],
                gsems[c % 2],
            )

        def start_scatter(c):
            return pltpu.async_copy(
                bufs[c % 2].at[pl.ds(0, sz(c))],
                out_hbm.at[pl.ds(base + c * CH, sz(c))],
                ssems[c % 2],
            )

        gh = {}
        sh = {}
        for c in range(n_chunks):
            if c >= 2:
                sh[c - 2].wait()
            gh[c] = start_gather(c)
            if c >= 1:
                gh[c - 1].wait()
                sh[c - 1] = start_scatter(c - 1)
        gh[n_chunks - 1].wait()
        sh[n_chunks - 1] = start_scatter(n_chunks - 1)
        if n_chunks >= 2:
            sh[n_chunks - 2].wait()
        sh[n_chunks - 1].wait()

    return gk(table, idx_p)[:B]


def _feast_attn(xn2, n, b, f, out_c, WmT, bm2, WoAllT, Eoc, SumM, bias2, act):
    rows = b * n
    NB = min(512, _cdiv(rows, 8) * 8)
    grid = (_cdiv(rows, NB),)

    def kern(xn_ref, wm_ref, bm_ref, woall_ref, eoc_ref, summ_ref,
             bias_ref, o_ref):
        xb = xn_ref[...]
        x0 = xb[:, 0:f]
        T = jnp.zeros((NB, _HEADS * out_c), jnp.float32)
        for k in range(_K):
            xk = xb[:, k * f:(k + 1) * f]
            lg = jnp.dot(xk - x0, wm_ref[...],
                         preferred_element_type=jnp.float32) + bm_ref[...]
            lg = lg - jnp.max(lg, axis=-1, keepdims=True)
            e = jnp.exp(lg)
            qk = e / jnp.sum(e, axis=-1, keepdims=True)
            yk = jnp.dot(xk, woall_ref[...],
                         preferred_element_type=jnp.float32)
            qbk = jnp.dot(qk, eoc_ref[...],
                          preferred_element_type=jnp.float32)
            T = T + qbk * yk
        out = jnp.dot(T, summ_ref[...],
                      preferred_element_type=jnp.float32) + bias_ref[...]
        gid = pl.program_id(0) * NB + lax.broadcasted_iota(jnp.int32, (NB, 1), 0)
        out = jnp.where((gid % n) == (n - 1), 0.0, out)
        if act:
            out = jnp.where(out > 0, out, jnp.exp(out) - 1.0)
        o_ref[...] = out

    return pl.pallas_call(
        kern,
        grid=grid,
        in_specs=[
            pl.BlockSpec((NB, _K * f), lambda i: (i, 0)),
            pl.BlockSpec(WmT.shape, lambda i: (0, 0)),
            pl.BlockSpec(bm2.shape, lambda i: (0, 0)),
            pl.BlockSpec(WoAllT.shape, lambda i: (0, 0)),
            pl.BlockSpec(Eoc.shape, lambda i: (0, 0)),
            pl.BlockSpec(SumM.shape, lambda i: (0, 0)),
            pl.BlockSpec(bias2.shape, lambda i: (0, 0)),
        ],
        out_specs=pl.BlockSpec((NB, out_c), lambda i: (i, 0)),
        out_shape=jax.ShapeDtypeStruct((rows, out_c), jnp.float32),
    )(xn2, WmT, bm2, WoAllT, Eoc, SumM, bias2)


def _matmul(A, B, bias2):
    M, Kd = A.shape
    _, N = B.shape
    BM = min(256, M)
    BN = min(512, N)
    BK = Kd if Kd <= 2560 else 2048
    grid = (_cdiv(M, BM), _cdiv(N, BN), _cdiv(Kd, BK))

    def kern(a_ref, b_ref, bias_ref, o_ref):
        @pl.when(pl.program_id(2) == 0)
        def _():
            o_ref[...] = jnp.zeros_like(o_ref)

        a = a_ref[...]
        if Kd % BK != 0:
            kbase = pl.program_id(2) * BK
            lane = lax.broadcasted_iota(jnp.int32, (BM, BK), 1)
            a = jnp.where(lane + kbase < Kd, a, 0.0)
        o_ref[...] += jnp.dot(a, b_ref[...],
                              preferred_element_type=jnp.float32)

        @pl.when(pl.program_id(2) == pl.num_programs(2) - 1)
        def _():
            o_ref[...] += bias_ref[...]

    return pl.pallas_call(
        kern,
        grid=grid,
        in_specs=[
            pl.BlockSpec((BM, BK), lambda i, j, k: (i, k)),
            pl.BlockSpec((BK, BN), lambda i, j, k: (k, j)),
            pl.BlockSpec((1, BN), lambda i, j, k: (0, j)),
        ],
        out_specs=pl.BlockSpec((BM, BN), lambda i, j, k: (i, j)),
        out_shape=jax.ShapeDtypeStruct((M, N), jnp.float32),
    )(A, B, bias2)


def _prep_conv(p, in_c, f_pad, out_c):
    Wm = p["Wm"]
    Wo = p["Wo"]
    WmT = jnp.pad(Wm, ((0, 0), (0, f_pad - in_c))).T
    WoAllT = jnp.pad(Wo.T, ((0, f_pad - in_c), (0, 0)))
    Eoc = jnp.kron(jnp.eye(_HEADS, dtype=jnp.float32),
                   jnp.ones((1, out_c), jnp.float32))
    SumM = jnp.kron(jnp.ones((_HEADS, 1), jnp.float32),
                    jnp.eye(out_c, dtype=jnp.float32))
    return WmT, p["bm"][None, :], WoAllT, Eoc, SumM, p["bias"][None, :]


def _feast_layer(h, Si, p, out_c, act):
    b, n, f = h.shape
    f_pad = max(16, f)
    in_c = f
    if f_pad != f:
        h = jnp.pad(h, ((0, 0), (0, 0), (0, f_pad - f)))
        f = f_pad
    WmT, bm2, WoAllT, Eoc, SumM, bias2 = _prep_conv(p, in_c, f_pad, out_c)
    table = h.reshape(b * n, f)
    offs = (jnp.arange(b, dtype=jnp.int32) * n)[:, None]
    idx = (Si.reshape(-1)[None, :] + offs).reshape(-1)
    xn = _sc_gather(table, idx)
    xn2 = xn.reshape(b * n, _K * f)
    out = _feast_attn(xn2, n, b, f, out_c, WmT, bm2, WoAllT, Eoc, SumM,
                      bias2, act)
    return out.reshape(b, n, out_c)


def _pool(L, h):
    b, m, f = h.shape
    p = L.shape[0]
    h2 = h.transpose(1, 0, 2).reshape(m, b * f)
    zb = jnp.zeros((1, b * f), jnp.float32)
    out2 = _matmul(L, h2, zb)
    return out2.reshape(p, b, f).transpose(1, 0, 2)


def kernel(x, S, D, U, theta):
    c = theta["convs"]
    b = x.shape[0]
    h = _feast_layer(x, S[0], c[0], 32, True)
    h = _pool(D[0], h)
    h = _feast_layer(h, S[1], c[1], 32, True)
    h = _pool(D[1], h)
    h = _feast_layer(h, S[2], c[2], 64, True)
    h = _pool(D[2], h)
    nf = h.shape[1] * h.shape[2]
    z = _matmul(h.reshape(b, nf), theta["We"].T, theta["be"][None, :])
    hd = _matmul(z, theta["Wd"].T, theta["bd"][None, :])
    h = hd.reshape(b, -1, 64)
    h = _pool(U[2], h)
    h = _feast_layer(h, S[2], c[3], 64, True)
    h = _pool(U[1], h)
    h = _feast_layer(h, S[1], c[4], 32, True)
    h = _pool(U[0], h)
    h = _feast_layer(h, S[0], c[5], 32, True)
    h = _feast_layer(h, S[0], c[6], 3, False)
    return h


# speedup vs baseline: 2.4207x; 1.1175x over previous
import functools

import jax
import jax.numpy as jnp
from jax import lax
from jax.experimental import pallas as pl
from jax.experimental.pallas import tpu as pltpu
from jax.experimental.pallas import tpu_sc as plsc

_HEADS = 10
_K = 10
_NWORK = 32


def _cdiv(a, b):
    return (a + b - 1) // b


def _sc_gather(table, idx):
    _, f = table.shape
    B = idx.shape[0]
    CH = 512 if f > 32 else 1024
    per_w = _cdiv(_cdiv(B, _NWORK), 8) * 8
    Bp = _NWORK * per_w
    n_chunks = _cdiv(per_w, CH)
    idx_p = jnp.pad(idx, (0, Bp - B))

    mesh = plsc.VectorSubcoreMesh(core_axis_name="c", subcore_axis_name="s")

    @functools.partial(
        pl.kernel,
        mesh=mesh,
        out_type=jax.ShapeDtypeStruct((Bp, f), jnp.float32),
        scratch_types=[
            pltpu.VMEM((per_w,), jnp.int32),
            pltpu.VMEM((CH, f), jnp.float32),
            pltpu.VMEM((CH, f), jnp.float32),
            pltpu.SemaphoreType.DMA,
            pltpu.SemaphoreType.DMA,
            pltpu.SemaphoreType.DMA,
            pltpu.SemaphoreType.DMA,
        ],
        compiler_params=pltpu.CompilerParams(use_tc_tiling_on_sc=False),
    )
    def gk(table_hbm, idx_hbm, out_hbm, idx_v, buf0, buf1, g0, g1, s0, s1):
        wid = lax.axis_index("s") * 2 + lax.axis_index("c")
        base = wid * per_w
        pltpu.sync_copy(idx_hbm.at[pl.ds(base, per_w)], idx_v)
        bufs = (buf0, buf1)
        gsems = (g0, g1)
        ssems = (s0, s1)

        def sz(c):
            return CH if (c + 1) * CH <= per_w else per_w - c * CH

        def start_gather(c):
            return pltpu.async_copy(
                table_hbm.at[idx_v.at[pl.ds(c * CH, sz(c))]],
                bufs[c % 2].at[pl.ds(0, sz(c))],
                gsems[c % 2],
            )

        def start_scatter(c):
            return pltpu.async_copy(
                bufs[c % 2].at[pl.ds(0, sz(c))],
                out_hbm.at[pl.ds(base + c * CH, sz(c))],
                ssems[c % 2],
            )

        gh = {}
        sh = {}
        for c in range(n_chunks):
            if c >= 2:
                sh[c - 2].wait()
            gh[c] = start_gather(c)
            if c >= 1:
                gh[c - 1].wait()
                sh[c - 1] = start_scatter(c - 1)
        gh[n_chunks - 1].wait()
        sh[n_chunks - 1] = start_scatter(n_chunks - 1)
        if n_chunks >= 2:
            sh[n_chunks - 2].wait()
        sh[n_chunks - 1].wait()

    return gk(table, idx_p)[:B]


def _feast_attn(xn2, n, b, f, out_c, WmT, bm2, WoAllT, Eoc, SumM, bias2, act):
    rows = b * n
    NB = min(1024, _cdiv(rows, 8) * 8)
    grid = (_cdiv(rows, NB),)

    def kern(xn_ref, wm_ref, bm_ref, woall_ref, eoc_ref, summ_ref,
             bias_ref, o_ref):
        xb = xn_ref[...]
        x0 = xb[:, 0:f]
        T = jnp.zeros((NB, _HEADS * out_c), jnp.float32)
        for k in range(_K):
            xk = xb[:, k * f:(k + 1) * f]
            lg = jnp.dot(xk - x0, wm_ref[...],
                         preferred_element_type=jnp.float32) + bm_ref[...]
            lg = lg - jnp.max(lg, axis=-1, keepdims=True)
            e = jnp.exp(lg)
            qk = e / jnp.sum(e, axis=-1, keepdims=True)
            yk = jnp.dot(xk, woall_ref[...],
                         preferred_element_type=jnp.float32)
            qbk = jnp.dot(qk, eoc_ref[...],
                          preferred_element_type=jnp.float32)
            T = T + qbk * yk
        out = jnp.dot(T, summ_ref[...],
                      preferred_element_type=jnp.float32) + bias_ref[...]
        gid = pl.program_id(0) * NB + lax.broadcasted_iota(jnp.int32, (NB, 1), 0)
        out = jnp.where((gid % n) == (n - 1), 0.0, out)
        if act:
            out = jnp.where(out > 0, out, jnp.exp(out) - 1.0)
        o_ref[...] = out

    return pl.pallas_call(
        kern,
        grid=grid,
        in_specs=[
            pl.BlockSpec((NB, _K * f), lambda i: (i, 0)),
            pl.BlockSpec(WmT.shape, lambda i: (0, 0)),
            pl.BlockSpec(bm2.shape, lambda i: (0, 0)),
            pl.BlockSpec(WoAllT.shape, lambda i: (0, 0)),
            pl.BlockSpec(Eoc.shape, lambda i: (0, 0)),
            pl.BlockSpec(SumM.shape, lambda i: (0, 0)),
            pl.BlockSpec(bias2.shape, lambda i: (0, 0)),
        ],
        out_specs=pl.BlockSpec((NB, out_c), lambda i: (i, 0)),
        out_shape=jax.ShapeDtypeStruct((rows, out_c), jnp.float32),
    )(xn2, WmT, bm2, WoAllT, Eoc, SumM, bias2)


def _matmul(A, B, bias2):
    M, Kd = A.shape
    _, N = B.shape
    BM = min(256, M)
    BN = min(512, N)
    BK = Kd if Kd <= 2560 else 2048
    grid = (_cdiv(M, BM), _cdiv(N, BN), _cdiv(Kd, BK))

    def kern(a_ref, b_ref, bias_ref, o_ref):
        @pl.when(pl.program_id(2) == 0)
        def _():
            o_ref[...] = jnp.zeros_like(o_ref)

        if Kd % BK != 0:
            @pl.when(pl.program_id(2) == pl.num_programs(2) - 1)
            def _():
                kbase = pl.program_id(2) * BK
                lane = lax.broadcasted_iota(jnp.int32, (BM, BK), 1)
                a = jnp.where(lane + kbase < Kd, a_ref[...], 0.0)
                o_ref[...] += jnp.dot(a, b_ref[...],
                                      preferred_element_type=jnp.float32)

            @pl.when(pl.program_id(2) < pl.num_programs(2) - 1)
            def _():
                o_ref[...] += jnp.dot(a_ref[...], b_ref[...],
                                      preferred_element_type=jnp.float32)
        else:
            o_ref[...] += jnp.dot(a_ref[...], b_ref[...],
                                  preferred_element_type=jnp.float32)

        @pl.when(pl.program_id(2) == pl.num_programs(2) - 1)
        def _():
            o_ref[...] += bias_ref[...]

    return pl.pallas_call(
        kern,
        grid=grid,
        in_specs=[
            pl.BlockSpec((BM, BK), lambda i, j, k: (i, k)),
            pl.BlockSpec((BK, BN), lambda i, j, k: (k, j)),
            pl.BlockSpec((1, BN), lambda i, j, k: (0, j)),
        ],
        out_specs=pl.BlockSpec((BM, BN), lambda i, j, k: (i, j)),
        out_shape=jax.ShapeDtypeStruct((M, N), jnp.float32),
    )(A, B, bias2)


def _prep_conv(p, in_c, f_pad, out_c):
    Wm = p["Wm"]
    Wo = p["Wo"]
    WmT = jnp.pad(Wm, ((0, 0), (0, f_pad - in_c))).T
    WoAllT = jnp.pad(Wo.T, ((0, f_pad - in_c), (0, 0)))
    Eoc = jnp.kron(jnp.eye(_HEADS, dtype=jnp.float32),
                   jnp.ones((1, out_c), jnp.float32))
    SumM = jnp.kron(jnp.ones((_HEADS, 1), jnp.float32),
                    jnp.eye(out_c, dtype=jnp.float32))
    return WmT, p["bm"][None, :], WoAllT, Eoc, SumM, p["bias"][None, :]


def _feast_layer(h, Si, p, out_c, act):
    b, n, f = h.shape
    f_pad = max(16, f)
    in_c = f
    if f_pad != f:
        h = jnp.pad(h, ((0, 0), (0, 0), (0, f_pad - f)))
        f = f_pad
    WmT, bm2, WoAllT, Eoc, SumM, bias2 = _prep_conv(p, in_c, f_pad, out_c)
    table = h.reshape(b * n, f)
    offs = (jnp.arange(b, dtype=jnp.int32) * n)[:, None]
    idx = (Si.reshape(-1)[None, :] + offs).reshape(-1)
    xn = _sc_gather(table, idx)
    xn2 = xn.reshape(b * n, _K * f)
    out = _feast_attn(xn2, n, b, f, out_c, WmT, bm2, WoAllT, Eoc, SumM,
                      bias2, act)
    return out.reshape(b, n, out_c)


def _pool(L, h):
    b, m, f = h.shape
    p = L.shape[0]
    h2 = h.transpose(1, 0, 2).reshape(m, b * f)
    zb = jnp.zeros((1, b * f), jnp.float32)
    out2 = _matmul(L, h2, zb)
    return out2.reshape(p, b, f).transpose(1, 0, 2)


def kernel(x, S, D, U, theta):
    c = theta["convs"]
    b = x.shape[0]
    h = _feast_layer(x, S[0], c[0], 32, True)
    h = _pool(D[0], h)
    h = _feast_layer(h, S[1], c[1], 32, True)
    h = _pool(D[1], h)
    h = _feast_layer(h, S[2], c[2], 64, True)
    h = _pool(D[2], h)
    nf = h.shape[1] * h.shape[2]
    z = _matmul(h.reshape(b, nf), theta["We"].T, theta["be"][None, :])
    hd = _matmul(z, theta["Wd"].T, theta["bd"][None, :])
    h = hd.reshape(b, -1, 64)
    h = _pool(U[2], h)
    h = _feast_layer(h, S[2], c[3], 64, True)
    h = _pool(U[1], h)
    h = _feast_layer(h, S[1], c[4], 32, True)
    h = _pool(U[0], h)
    h = _feast_layer(h, S[0], c[5], 32, True)
    h = _feast_layer(h, S[0], c[6], 3, False)
    return h


# speedup vs baseline: 2.5980x; 1.0733x over previous
import functools

import jax
import jax.numpy as jnp
from jax import lax
from jax.experimental import pallas as pl
from jax.experimental.pallas import tpu as pltpu
from jax.experimental.pallas import tpu_sc as plsc

_HEADS = 10
_K = 10
_NWORK = 32


def _cdiv(a, b):
    return (a + b - 1) // b


def _sc_gather(table, idx):
    _, f = table.shape
    B = idx.shape[0]
    CH = 512 if f > 32 else 1024
    per_w = _cdiv(_cdiv(B, _NWORK), 8) * 8
    Bp = _NWORK * per_w
    n_chunks = _cdiv(per_w, CH)
    idx_p = jnp.pad(idx, (0, Bp - B))

    mesh = plsc.VectorSubcoreMesh(core_axis_name="c", subcore_axis_name="s")

    @functools.partial(
        pl.kernel,
        mesh=mesh,
        out_type=jax.ShapeDtypeStruct((Bp, f), jnp.float32),
        scratch_types=[
            pltpu.VMEM((per_w,), jnp.int32),
            pltpu.VMEM((CH, f), jnp.float32),
            pltpu.VMEM((CH, f), jnp.float32),
            pltpu.SemaphoreType.DMA,
            pltpu.SemaphoreType.DMA,
            pltpu.SemaphoreType.DMA,
            pltpu.SemaphoreType.DMA,
        ],
        compiler_params=pltpu.CompilerParams(use_tc_tiling_on_sc=False),
    )
    def gk(table_hbm, idx_hbm, out_hbm, idx_v, buf0, buf1, g0, g1, s0, s1):
        wid = lax.axis_index("s") * 2 + lax.axis_index("c")
        base = wid * per_w
        pltpu.sync_copy(idx_hbm.at[pl.ds(base, per_w)], idx_v)
        bufs = (buf0, buf1)
        gsems = (g0, g1)
        ssems = (s0, s1)

        def sz(c):
            return CH if (c + 1) * CH <= per_w else per_w - c * CH

        def start_gather(c):
            return pltpu.async_copy(
                table_hbm.at[idx_v.at[pl.ds(c * CH, sz(c))]],
                bufs[c % 2].at[pl.ds(0, sz(c))],
                gsems[c % 2],
            )

        def start_scatter(c):
            return pltpu.async_copy(
                bufs[c % 2].at[pl.ds(0, sz(c))],
                out_hbm.at[pl.ds(base + c * CH, sz(c))],
                ssems[c % 2],
            )

        gh = {}
        sh = {}
        for c in range(n_chunks):
            if c >= 2:
                sh[c - 2].wait()
            gh[c] = start_gather(c)
            if c >= 1:
                gh[c - 1].wait()
                sh[c - 1] = start_scatter(c - 1)
        gh[n_chunks - 1].wait()
        sh[n_chunks - 1] = start_scatter(n_chunks - 1)
        if n_chunks >= 2:
            sh[n_chunks - 2].wait()
        sh[n_chunks - 1].wait()

    return gk(table, idx_p)[:B]


def _feast_attn(xn2, n, b, f, out_c, WmT, bm2, WoAllT, Eoc, SumM, bias2, act):
    rows = b * n
    NB = min(2048, _cdiv(rows, 8) * 8)
    grid = (_cdiv(rows, NB),)

    def kern(xn_ref, wm_ref, bm_ref, woall_ref, eoc_ref, summ_ref,
             bias_ref, o_ref):
        xb = xn_ref[...]
        x0 = xb[:, 0:f]
        T = jnp.zeros((NB, _HEADS * out_c), jnp.float32)
        for k in range(_K):
            xk = xb[:, k * f:(k + 1) * f]
            lg = jnp.dot(xk - x0, wm_ref[...],
                         preferred_element_type=jnp.float32) + bm_ref[...]
            lg = lg - jnp.max(lg, axis=-1, keepdims=True)
            e = jnp.exp(lg)
            qk = e / jnp.sum(e, axis=-1, keepdims=True)
            yk = jnp.dot(xk, woall_ref[...],
                         preferred_element_type=jnp.float32)
            qbk = jnp.dot(qk, eoc_ref[...],
                          preferred_element_type=jnp.float32)
            T = T + qbk * yk
        out = jnp.dot(T, summ_ref[...],
                      preferred_element_type=jnp.float32) + bias_ref[...]
        gid = pl.program_id(0) * NB + lax.broadcasted_iota(jnp.int32, (NB, 1), 0)
        out = jnp.where((gid % n) == (n - 1), 0.0, out)
        if act:
            out = jnp.where(out > 0, out, jnp.exp(out) - 1.0)
        o_ref[...] = out

    return pl.pallas_call(
        kern,
        grid=grid,
        in_specs=[
            pl.BlockSpec((NB, _K * f), lambda i: (i, 0)),
            pl.BlockSpec(WmT.shape, lambda i: (0, 0)),
            pl.BlockSpec(bm2.shape, lambda i: (0, 0)),
            pl.BlockSpec(WoAllT.shape, lambda i: (0, 0)),
            pl.BlockSpec(Eoc.shape, lambda i: (0, 0)),
            pl.BlockSpec(SumM.shape, lambda i: (0, 0)),
            pl.BlockSpec(bias2.shape, lambda i: (0, 0)),
        ],
        out_specs=pl.BlockSpec((NB, out_c), lambda i: (i, 0)),
        out_shape=jax.ShapeDtypeStruct((rows, out_c), jnp.float32),
    )(xn2, WmT, bm2, WoAllT, Eoc, SumM, bias2)


def _matmul(A, B, bias2):
    M, Kd = A.shape
    _, N = B.shape
    BM = min(512, M)
    BN = min(512, N)
    BK = Kd if Kd <= 2560 else 2048
    grid = (_cdiv(M, BM), _cdiv(N, BN), _cdiv(Kd, BK))

    def kern(a_ref, b_ref, bias_ref, o_ref):
        @pl.when(pl.program_id(2) == 0)
        def _():
            o_ref[...] = jnp.zeros_like(o_ref)

        if Kd % BK != 0:
            @pl.when(pl.program_id(2) == pl.num_programs(2) - 1)
            def _():
                kbase = pl.program_id(2) * BK
                lane = lax.broadcasted_iota(jnp.int32, (BM, BK), 1)
                a = jnp.where(lane + kbase < Kd, a_ref[...], 0.0)
                o_ref[...] += jnp.dot(a, b_ref[...],
                                      preferred_element_type=jnp.float32)

            @pl.when(pl.program_id(2) < pl.num_programs(2) - 1)
            def _():
                o_ref[...] += jnp.dot(a_ref[...], b_ref[...],
                                      preferred_element_type=jnp.float32)
        else:
            o_ref[...] += jnp.dot(a_ref[...], b_ref[...],
                                  preferred_element_type=jnp.float32)

        @pl.when(pl.program_id(2) == pl.num_programs(2) - 1)
        def _():
            o_ref[...] += bias_ref[...]

    return pl.pallas_call(
        kern,
        grid=grid,
        in_specs=[
            pl.BlockSpec((BM, BK), lambda i, j, k: (i, k)),
            pl.BlockSpec((BK, BN), lambda i, j, k: (k, j)),
            pl.BlockSpec((1, BN), lambda i, j, k: (0, j)),
        ],
        out_specs=pl.BlockSpec((BM, BN), lambda i, j, k: (i, j)),
        out_shape=jax.ShapeDtypeStruct((M, N), jnp.float32),
    )(A, B, bias2)


def _prep_conv(p, in_c, f_pad, out_c):
    Wm = p["Wm"]
    Wo = p["Wo"]
    WmT = jnp.pad(Wm, ((0, 0), (0, f_pad - in_c))).T
    WoAllT = jnp.pad(Wo.T, ((0, f_pad - in_c), (0, 0)))
    Eoc = jnp.kron(jnp.eye(_HEADS, dtype=jnp.float32),
                   jnp.ones((1, out_c), jnp.float32))
    SumM = jnp.kron(jnp.ones((_HEADS, 1), jnp.float32),
                    jnp.eye(out_c, dtype=jnp.float32))
    return WmT, p["bm"][None, :], WoAllT, Eoc, SumM, p["bias"][None, :]


def _feast_layer(h, Si, p, out_c, act):
    b, n, f = h.shape
    f_pad = max(16, f)
    in_c = f
    if f_pad != f:
        h = jnp.pad(h, ((0, 0), (0, 0), (0, f_pad - f)))
        f = f_pad
    WmT, bm2, WoAllT, Eoc, SumM, bias2 = _prep_conv(p, in_c, f_pad, out_c)
    table = h.reshape(b * n, f)
    offs = (jnp.arange(b, dtype=jnp.int32) * n)[:, None]
    idx = (Si.reshape(-1)[None, :] + offs).reshape(-1)
    xn = _sc_gather(table, idx)
    xn2 = xn.reshape(b * n, _K * f)
    out = _feast_attn(xn2, n, b, f, out_c, WmT, bm2, WoAllT, Eoc, SumM,
                      bias2, act)
    return out.reshape(b, n, out_c)


def _pool(L, h):
    b, m, f = h.shape
    p = L.shape[0]
    h2 = h.transpose(1, 0, 2).reshape(m, b * f)
    zb = jnp.zeros((1, b * f), jnp.float32)
    out2 = _matmul(L, h2, zb)
    return out2.reshape(p, b, f).transpose(1, 0, 2)


def kernel(x, S, D, U, theta):
    c = theta["convs"]
    b = x.shape[0]
    h = _feast_layer(x, S[0], c[0], 32, True)
    h = _pool(D[0], h)
    h = _feast_layer(h, S[1], c[1], 32, True)
    h = _pool(D[1], h)
    h = _feast_layer(h, S[2], c[2], 64, True)
    h = _pool(D[2], h)
    nf = h.shape[1] * h.shape[2]
    z = _matmul(h.reshape(b, nf), theta["We"].T, theta["be"][None, :])
    hd = _matmul(z, theta["Wd"].T, theta["bd"][None, :])
    h = hd.reshape(b, -1, 64)
    h = _pool(U[2], h)
    h = _feast_layer(h, S[2], c[3], 64, True)
    h = _pool(U[1], h)
    h = _feast_layer(h, S[1], c[4], 32, True)
    h = _pool(U[0], h)
    h = _feast_layer(h, S[0], c[5], 32, True)
    h = _feast_layer(h, S[0], c[6], 3, False)
    return h
